# baseline (speedup 1.0000x reference)
import jax
import jax.numpy as jnp
from jax import lax
from jax.experimental import pallas as pl
from jax.experimental.pallas import tpu as pltpu

VOCAB = 100000
EMBED = 128
CTX = 20
HID = 128
NB = 50
TILE = VOCAB // NB


def _body(idx_ref, emb_ref, w1_ref, b1_ref, w2_ref, b2_ref, out_ref,
          gbuf, h_s, sem):
    i = pl.program_id(0)

    @pl.when(i == 0)
    def _():
        copies = [
            pltpu.make_async_copy(
                emb_ref.at[pl.ds(idx_ref[t], 1), :],
                gbuf.at[pl.ds(t, 1), :],
                sem,
            )
            for t in range(CTX)
        ]
        for c in copies:
            c.start()
        for c in copies:
            c.wait()
        acc = jnp.zeros((1, HID), jnp.float32)
        for t in range(CTX):
            acc += lax.dot_general(
                gbuf[t:t + 1, :],
                w1_ref[:, t * EMBED:(t + 1) * EMBED],
                (((1,), (1,)), ((), ())),
                preferred_element_type=jnp.float32)
        h_s[...] = jnp.maximum(acc + b1_ref[...], 0.0)

    @pl.when(i < NB)
    def _():
        tile = lax.dot_general(h_s[...], w2_ref[...],
                               (((1,), (1,)), ((), ())),
                               preferred_element_type=jnp.float32)
        out_ref[i] = tile + b2_ref[0]

    @pl.when(i == NB)
    def _():
        allv = out_ref[...]
        m = jnp.max(allv)
        s = jnp.sum(jnp.exp(allv - m))
        out_ref[...] = allv - (m + jnp.log(s))


def kernel(inputs, emb, W1, b1, W2, b2):
    idx = inputs.astype(jnp.int32)

    grid_spec = pltpu.PrefetchScalarGridSpec(
        num_scalar_prefetch=1,
        grid=(NB + 1,),
        in_specs=[
            pl.BlockSpec(memory_space=pltpu.HBM),
            pl.BlockSpec((HID, CTX * EMBED), lambda i, s: (0, 0)),
            pl.BlockSpec((1, HID), lambda i, s: (0, 0)),
            pl.BlockSpec((TILE, HID),
                         lambda i, s: (jnp.minimum(i, NB - 1), 0)),
            pl.BlockSpec((1, 1, TILE),
                         lambda i, s: (jnp.minimum(i, NB - 1), 0, 0)),
        ],
        out_specs=pl.BlockSpec((NB, 1, TILE), lambda i, s: (0, 0, 0)),
        scratch_shapes=[
            pltpu.VMEM((CTX, EMBED), jnp.float32),
            pltpu.VMEM((1, HID), jnp.float32),
            pltpu.SemaphoreType.DMA,
        ],
    )

    log_probs = pl.pallas_call(
        _body,
        grid_spec=grid_spec,
        out_shape=jax.ShapeDtypeStruct((NB, 1, TILE), jnp.float32),
    )(idx, emb, W1, b1.reshape(1, HID), W2, b2.reshape(NB, 1, TILE))

    return log_probs.reshape(1, VOCAB)


# speedup vs baseline: 1.7435x; 1.7435x over previous
import jax
import jax.numpy as jnp
from jax import lax
from jax.experimental import pallas as pl
from jax.experimental.pallas import tpu as pltpu

VOCAB = 100000
EMBED = 128
CTX = 20
HID = 128
NB = 10
TILE = VOCAB // NB


def _body(idx_ref, emb_ref, w1_ref, b1_ref, w2_ref, b2_ref, out_ref,
          gbuf, h_s, sem):
    i = pl.program_id(0)

    @pl.when(i == 0)
    def _():
        copies = [
            pltpu.make_async_copy(
                emb_ref.at[pl.ds(idx_ref[t], 1), :],
                gbuf.at[pl.ds(t, 1), :],
                sem,
            )
            for t in range(CTX)
        ]
        for c in copies:
            c.start()
        for c in copies:
            c.wait()
        acc = jnp.zeros((1, HID), jnp.float32)
        for t in range(CTX):
            acc += lax.dot_general(
                gbuf[t:t + 1, :],
                w1_ref[:, t * EMBED:(t + 1) * EMBED],
                (((1,), (1,)), ((), ())),
                preferred_element_type=jnp.float32)
        h_s[...] = jnp.maximum(acc + b1_ref[...], 0.0)

    @pl.when(i < NB)
    def _():
        tile = lax.dot_general(h_s[...], w2_ref[...],
                               (((1,), (1,)), ((), ())),
                               preferred_element_type=jnp.float32)
        out_ref[i] = tile + b2_ref[0]

    @pl.when(i == NB)
    def _():
        allv = out_ref[...]
        m = jnp.max(allv)
        s = jnp.sum(jnp.exp(allv - m))
        out_ref[...] = allv - (m + jnp.log(s))


def kernel(inputs, emb, W1, b1, W2, b2):
    idx = inputs.astype(jnp.int32)

    grid_spec = pltpu.PrefetchScalarGridSpec(
        num_scalar_prefetch=1,
        grid=(NB + 1,),
        in_specs=[
            pl.BlockSpec(memory_space=pltpu.HBM),
            pl.BlockSpec((HID, CTX * EMBED), lambda i, s: (0, 0)),
            pl.BlockSpec((1, HID), lambda i, s: (0, 0)),
            pl.BlockSpec((TILE, HID),
                         lambda i, s: (jnp.minimum(i, NB - 1), 0)),
            pl.BlockSpec((1, 1, TILE),
                         lambda i, s: (jnp.minimum(i, NB - 1), 0, 0)),
        ],
        out_specs=pl.BlockSpec((NB, 1, TILE), lambda i, s: (0, 0, 0)),
        scratch_shapes=[
            pltpu.VMEM((CTX, EMBED), jnp.float32),
            pltpu.VMEM((1, HID), jnp.float32),
            pltpu.SemaphoreType.DMA,
        ],
    )

    log_probs = pl.pallas_call(
        _body,
        grid_spec=grid_spec,
        out_shape=jax.ShapeDtypeStruct((NB, 1, TILE), jnp.float32),
    )(idx, emb, W1, b1.reshape(1, HID), W2, b2.reshape(NB, 1, TILE))

    return log_probs.reshape(1, VOCAB)


# speedup vs baseline: 1.7684x; 1.0143x over previous
import jax
import jax.numpy as jnp
from jax import lax
from jax.experimental import pallas as pl
from jax.experimental.pallas import tpu as pltpu

VOCAB = 100000
EMBED = 128
CTX = 20
HID = 128
NB = 5
TILE = VOCAB // NB


def _body(idx_ref, emb_ref, w1_ref, b1_ref, w2_ref, b2_ref, out_ref,
          gbuf, h_s, sem):
    i = pl.program_id(0)

    @pl.when(i == 0)
    def _():
        copies = [
            pltpu.make_async_copy(
                emb_ref.at[pl.ds(idx_ref[t], 1), :],
                gbuf.at[pl.ds(t, 1), :],
                sem,
            )
            for t in range(CTX)
        ]
        for c in copies:
            c.start()
        for c in copies:
            c.wait()
        acc = jnp.zeros((1, HID), jnp.float32)
        for t in range(CTX):
            acc += lax.dot_general(
                gbuf[t:t + 1, :],
                w1_ref[:, t * EMBED:(t + 1) * EMBED],
                (((1,), (1,)), ((), ())),
                preferred_element_type=jnp.float32)
        h_s[...] = jnp.maximum(acc + b1_ref[...], 0.0)

    @pl.when(i < NB)
    def _():
        tile = lax.dot_general(h_s[...], w2_ref[...],
                               (((1,), (1,)), ((), ())),
                               preferred_element_type=jnp.float32)
        out_ref[i] = tile + b2_ref[0]

    @pl.when(i == NB)
    def _():
        allv = out_ref[...]
        m = jnp.max(allv)
        s = jnp.sum(jnp.exp(allv - m))
        out_ref[...] = allv - (m + jnp.log(s))


def kernel(inputs, emb, W1, b1, W2, b2):
    idx = inputs.astype(jnp.int32)

    grid_spec = pltpu.PrefetchScalarGridSpec(
        num_scalar_prefetch=1,
        grid=(NB + 1,),
        in_specs=[
            pl.BlockSpec(memory_space=pltpu.HBM),
            pl.BlockSpec((HID, CTX * EMBED), lambda i, s: (0, 0)),
            pl.BlockSpec((1, HID), lambda i, s: (0, 0)),
            pl.BlockSpec((TILE, HID),
                         lambda i, s: (jnp.minimum(i, NB - 1), 0)),
            pl.BlockSpec((1, 1, TILE),
                         lambda i, s: (jnp.minimum(i, NB - 1), 0, 0)),
        ],
        out_specs=pl.BlockSpec((NB, 1, TILE), lambda i, s: (0, 0, 0)),
        scratch_shapes=[
            pltpu.VMEM((CTX, EMBED), jnp.float32),
            pltpu.VMEM((1, HID), jnp.float32),
            pltpu.SemaphoreType.DMA,
        ],
    )

    log_probs = pl.pallas_call(
        _body,
        grid_spec=grid_spec,
        out_shape=jax.ShapeDtypeStruct((NB, 1, TILE), jnp.float32),
    )(idx, emb, W1, b1.reshape(1, HID), W2, b2.reshape(NB, 1, TILE))

    return log_probs.reshape(1, VOCAB)
